# baseline (speedup 1.0000x reference)
import functools

import jax
import jax.numpy as jnp
import numpy as np
from jax import lax
from jax.experimental import pallas as pl
from jax.experimental.pallas import tpu as pltpu
from jax.experimental.pallas import tpu_sc as plsc

D = 32
NC = 2
NS = 16
NW = NC * NS
AW = 128
BC = 8
SUB = 128
SCALE = np.float32(np.sqrt(np.float32(32.0)))


@functools.lru_cache(maxsize=None)
def _make_kernel(A: int, B: int):
  assert A == NW * AW and B % BC == 0
  G = B // BC
  CT = BC * AW

  mesh = plsc.VectorSubcoreMesh(core_axis_name="c", subcore_axis_name="s")

  @functools.partial(
      pl.kernel,
      out_type=jax.ShapeDtypeStruct((B, D, A), jnp.float32),
      mesh=mesh,
      scratch_types=[
          pltpu.VMEM((BC, AW), jnp.int32),
          pltpu.VMEM((CT, D), jnp.float32),
          pltpu.VMEM((BC, D, AW), jnp.float32),
          pltpu.SemaphoreType.DMA,
      ],
      compiler_params=pltpu.CompilerParams(
          use_tc_tiling_on_sc=False, needs_layout_passes=False),
  )
  def emb_kernel(tokens_hbm, table_hbm, out_hbm, idx_v, rows_v, tr_v, sem):
    wid = lax.axis_index("s") * NC + lax.axis_index("c")
    a0 = wid * AW
    iota = lax.iota(jnp.int32, 16)
    jvecs = [iota + h * 16 for h in range(D // 16)]

    @pl.loop(0, G)
    def chunk_loop(g):
      b0 = g * BC
      pltpu.sync_copy(
          tokens_hbm.at[pl.ds(b0, BC), pl.ds(a0, AW)], idx_v)

      gathers = [
          pltpu.async_copy(
              table_hbm.at[idx_v.at[b]],
              rows_v.at[pl.ds(b * AW, AW)],
              sem,
          )
          for b in range(BC)
      ]
      for cp in gathers:
        cp.wait()

      for b in range(BC):
        @pl.loop(0, AW, unroll=4)
        def tr_loop(a):
          avec = jnp.full((16,), a, jnp.int32)
          for h in range(D // 16):
            v = rows_v[b * AW + a, pl.ds(h * 16, 16)] * SCALE
            plsc.store_scatter(tr_v.at[b], [jvecs[h], avec], v)

      pltpu.sync_copy(
          tr_v, out_hbm.at[pl.ds(b0, BC), pl.ds(0, D), pl.ds(a0, AW)])

  return emb_kernel


@jax.jit
def kernel(tokens, table):
  A, B = tokens.shape
  tokens_t = jnp.swapaxes(tokens, 0, 1)
  out_t = _make_kernel(A, B)(tokens_t, table)
  return jnp.transpose(out_t, (2, 0, 1))


# speedup vs baseline: 1.3383x; 1.3383x over previous
import functools

import jax
import jax.numpy as jnp
import numpy as np
from jax import lax
from jax.experimental import pallas as pl
from jax.experimental.pallas import tpu as pltpu
from jax.experimental.pallas import tpu_sc as plsc

D = 32
NC = 2
NS = 16
NW = NC * NS
AW = 128
BC = 8
SUB = 128
SCALE = np.float32(np.sqrt(np.float32(32.0)))


@functools.lru_cache(maxsize=None)
def _make_kernel(A: int, B: int):
  assert A == NW * AW and B % BC == 0
  G = B // BC
  CT = BC * AW

  mesh = plsc.VectorSubcoreMesh(core_axis_name="c", subcore_axis_name="s")

  @functools.partial(
      pl.kernel,
      out_type=jax.ShapeDtypeStruct((B, D, A), jnp.float32),
      mesh=mesh,
      scratch_types=[
          pltpu.VMEM((BC, AW), jnp.int32),
          pltpu.VMEM((CT, D), jnp.float32),
          pltpu.VMEM((BC, D, AW + 1), jnp.float32),
          pltpu.SemaphoreType.DMA,
      ],
      compiler_params=pltpu.CompilerParams(
          use_tc_tiling_on_sc=False, needs_layout_passes=False),
  )
  def emb_kernel(tokens_hbm, table_hbm, out_hbm, idx_v, rows_v, tr_v, sem):
    wid = lax.axis_index("s") * NC + lax.axis_index("c")
    a0 = wid * AW
    iota = lax.iota(jnp.int32, 16)
    jvecs = [iota + h * 16 for h in range(D // 16)]

    @pl.loop(0, G)
    def chunk_loop(g):
      b0 = g * BC
      pltpu.sync_copy(
          tokens_hbm.at[pl.ds(b0, BC), pl.ds(a0, AW)], idx_v)

      gathers = [
          pltpu.async_copy(
              table_hbm.at[idx_v.at[b]],
              rows_v.at[pl.ds(b * AW, AW)],
              sem,
          )
          for b in range(BC)
      ]
      for cp in gathers:
        cp.wait()

      for b in range(BC):
        @pl.loop(0, AW, unroll=8)
        def tr_loop(a):
          avec = jnp.full((16,), a, jnp.int32)
          for h in range(D // 16):
            v = rows_v[b * AW + a, pl.ds(h * 16, 16)] * SCALE
            plsc.store_scatter(tr_v.at[b], [jvecs[h], avec], v)

      pltpu.sync_copy(
          tr_v.at[pl.ds(0, BC), pl.ds(0, D), pl.ds(0, AW)],
          out_hbm.at[pl.ds(b0, BC), pl.ds(0, D), pl.ds(a0, AW)])

  return emb_kernel


@jax.jit
def kernel(tokens, table):
  A, B = tokens.shape
  tokens_t = jnp.swapaxes(tokens, 0, 1)
  out_t = _make_kernel(A, B)(tokens_t, table)
  return jnp.transpose(out_t, (2, 0, 1))


# speedup vs baseline: 1.5063x; 1.1255x over previous
import functools

import jax
import jax.numpy as jnp
import numpy as np
from jax import lax
from jax.experimental import pallas as pl
from jax.experimental.pallas import tpu as pltpu
from jax.experimental.pallas import tpu_sc as plsc

D = 32
NC = 2
NS = 16
NW = NC * NS
AW = 128
BC = 8
SUB = 128
SCALE = np.float32(np.sqrt(np.float32(32.0)))


@functools.lru_cache(maxsize=None)
def _make_kernel(A: int, B: int):
  assert A == NW * AW and B % BC == 0
  G = B // BC
  CT = BC * AW

  mesh = plsc.VectorSubcoreMesh(core_axis_name="c", subcore_axis_name="s")

  BB, AB = B // BC, A // AW

  @functools.partial(
      pl.kernel,
      out_type=jax.ShapeDtypeStruct((B, D // 8, AB, 8, AW), jnp.float32),
      mesh=mesh,
      scratch_types=[
          pltpu.VMEM((BC, AW), jnp.int32),
          pltpu.VMEM((CT, D), jnp.float32),
          pltpu.VMEM((BC, D // 8, 8, AW + 1), jnp.float32),
          pltpu.SemaphoreType.DMA,
      ],
      compiler_params=pltpu.CompilerParams(
          use_tc_tiling_on_sc=False, needs_layout_passes=False),
  )
  def emb_kernel(tokens_hbm, table_hbm, out_hbm, idx_v, rows_v, tr_v, sem):
    wid = lax.axis_index("s") * NC + lax.axis_index("c")
    a0 = wid * AW
    iota = lax.iota(jnp.int32, 16)
    jbvecs = [(iota + h * 16) // 8 for h in range(D // 16)]
    j8vecs = [(iota + h * 16) % 8 for h in range(D // 16)]

    @pl.loop(0, G)
    def chunk_loop(g):
      b0 = g * BC
      pltpu.sync_copy(tokens_hbm.at[g, wid], idx_v)

      gathers = [
          pltpu.async_copy(
              table_hbm.at[idx_v.at[b]],
              rows_v.at[pl.ds(b * AW, AW)],
              sem,
          )
          for b in range(BC)
      ]
      for cp in gathers:
        cp.wait()

      for b in range(BC):
        @pl.loop(0, AW, unroll=8)
        def tr_loop(a):
          avec = jnp.full((16,), a, jnp.int32)
          for h in range(D // 16):
            v = rows_v[b * AW + a, pl.ds(h * 16, 16)] * SCALE
            plsc.store_scatter(
                tr_v.at[b], [jbvecs[h], j8vecs[h], avec], v)

      pltpu.sync_copy(
          tr_v.at[pl.ds(0, BC), pl.ds(0, D // 8), pl.ds(0, 8), pl.ds(0, AW)],
          out_hbm.at[pl.ds(b0, BC), pl.ds(0, D // 8), wid,
                     pl.ds(0, 8), pl.ds(0, AW)])

  return emb_kernel


@jax.jit
def kernel(tokens, table):
  A, B = tokens.shape
  tokens4d = jnp.transpose(
      tokens.reshape(A // AW, AW, B // BC, BC), (2, 0, 3, 1))
  out5d = _make_kernel(A, B)(tokens4d, table)
  out = jnp.transpose(out5d, (2, 4, 0, 1, 3)).reshape(A, B, D)
  return out


# speedup vs baseline: 1.5759x; 1.0462x over previous
import functools

import jax
import jax.numpy as jnp
import numpy as np
from jax import lax
from jax.experimental import pallas as pl
from jax.experimental.pallas import tpu as pltpu
from jax.experimental.pallas import tpu_sc as plsc

D = 32
NC = 2
NS = 16
NW = NC * NS
AW = 128
BT = 8
BC = 4
SUB = 128
SCALE = np.float32(np.sqrt(np.float32(32.0)))


@functools.lru_cache(maxsize=None)
def _make_kernel(A: int, B: int):
  assert A == NW * AW and B % BC == 0 and BT % BC == 0
  G = B // BC
  CT = BC * AW
  BB, AB = B // BT, A // AW
  assert G % 2 == 0

  mesh = plsc.VectorSubcoreMesh(core_axis_name="c", subcore_axis_name="s")

  @functools.partial(
      pl.kernel,
      out_type=jax.ShapeDtypeStruct((B, D // 8, AB, 8, AW), jnp.float32),
      mesh=mesh,
      scratch_types=[
          [pltpu.VMEM((BC, AW), jnp.int32) for _ in range(2)],
          [pltpu.VMEM((CT, D), jnp.float32) for _ in range(2)],
          [pltpu.VMEM((BC, D // 8, 8, AW + 1), jnp.float32) for _ in range(2)],
          [pltpu.SemaphoreType.DMA for _ in range(2)],
          [pltpu.SemaphoreType.DMA for _ in range(2)],
      ],
      compiler_params=pltpu.CompilerParams(
          use_tc_tiling_on_sc=False, needs_layout_passes=False),
  )
  def emb_kernel(tokens_hbm, table_hbm, out_hbm, idx_v, rows_v, tr_v,
                 gsem, osem):
    wid = lax.axis_index("s") * NC + lax.axis_index("c")
    iota = lax.iota(jnp.int32, 16)
    jbvecs = [(iota + h * 16) // 8 for h in range(D // 16)]
    j8vecs = [(iota + h * 16) % 8 for h in range(D // 16)]

    def fire(g, p):
      bb = g // (BT // BC)
      bs = (g % (BT // BC)) * BC
      pltpu.sync_copy(
          tokens_hbm.at[bb, wid, pl.ds(bs, BC), pl.ds(0, AW)], idx_v[p])
      for s in range(BC):
        pltpu.async_copy(
            table_hbm.at[idx_v[p].at[s]],
            rows_v[p].at[pl.ds(s * AW, AW)],
            gsem[p],
        )

    def wait_gathers(p):
      for s in range(BC):
        pltpu.make_async_copy(
            table_hbm.at[idx_v[p].at[s]],
            rows_v[p].at[pl.ds(s * AW, AW)],
            gsem[p],
        ).wait()

    def out_slice(g):
      return out_hbm.at[pl.ds(g * BC, BC), pl.ds(0, D // 8), wid,
                        pl.ds(0, 8), pl.ds(0, AW)]

    def tr_slice(p):
      return tr_v[p].at[pl.ds(0, BC), pl.ds(0, D // 8), pl.ds(0, 8),
                        pl.ds(0, AW)]

    def step(g, p):
      @pl.when(g + 1 < G)
      def _():
        fire(g + 1, 1 - p)

      wait_gathers(p)

      @pl.when(g >= 2)
      def _():
        pltpu.make_async_copy(tr_slice(p), out_slice(0), osem[p]).wait()

      for b in range(BC):
        @pl.loop(0, AW, unroll=8)
        def tr_loop(a):
          avec = jnp.full((16,), a, jnp.int32)
          for h in range(D // 16):
            v = rows_v[p][b * AW + a, pl.ds(h * 16, 16)] * SCALE
            plsc.store_scatter(
                tr_v[p].at[b], [jbvecs[h], j8vecs[h], avec], v)

      pltpu.async_copy(tr_slice(p), out_slice(g), osem[p])

    fire(0, 0)

    @pl.loop(0, G, step=2)
    def ring(q):
      step(q, 0)
      step(q + 1, 1)

    for p in range(2):
      pltpu.make_async_copy(tr_slice(p), out_slice(0), osem[p]).wait()

  return emb_kernel


@jax.jit
def kernel(tokens, table):
  A, B = tokens.shape
  tokens4d = jnp.transpose(
      tokens.reshape(A // AW, AW, B // BT, BT), (2, 0, 3, 1))
  out5d = _make_kernel(A, B)(tokens4d, table)
  out = jnp.transpose(out5d, (2, 4, 0, 1, 3)).reshape(A, B, D)
  return out
